# Initial kernel scaffold; baseline (speedup 1.0000x reference)
import jax
import jax.numpy as jnp
from jax.experimental import pallas as pl


def kernel(x, edge_index, W1, b1, W2, b2, Wh, bh, Wo, bo):
    raise NotImplementedError("write your pallas kernel here")


# speedup vs baseline: 5.1763x; 5.1763x over previous
import functools

import jax
import jax.numpy as jnp
from jax import lax
from jax.experimental import pallas as pl
from jax.experimental.pallas import tpu as pltpu
from jax.experimental.pallas import tpu_sc as plsc

N = 10000
E = 320000
D = 128
C = 10

NC = 2
NS = 16
NW = NC * NS
EPT = E // NW
CH = 80
NCHUNK = EPT // CH
NP = 10240
ZROWS = 128
RPS = NP // NS


def _agg_body(h_hbm, src_hbm, dst_hbm, out_hbm, idx_s, idx_d, rows, zbuf, agg, sem):
    cid = lax.axis_index("c")
    sid = lax.axis_index("s")
    wid = cid * NS + sid

    def zrow(i, carry):
        for j in range(D // 16):
            zbuf[i, pl.ds(j * 16, 16)] = jnp.zeros((16,), jnp.float32)
        return carry

    lax.fori_loop(0, ZROWS, zrow, 0)
    for k in range(RPS // ZROWS):
        pltpu.sync_copy(zbuf, agg.at[pl.ds(sid * RPS + k * ZROWS, ZROWS)])
    plsc.subcore_barrier()

    def chunk(i, carry):
        base = wid * EPT + i * CH
        pltpu.sync_copy(src_hbm.at[pl.ds(base, CH)], idx_s)
        pltpu.sync_copy(dst_hbm.at[pl.ds(base, CH)], idx_d)
        pltpu.async_copy(h_hbm.at[idx_s], rows, sem).wait()
        pltpu.sync_copy(rows, agg.at[idx_d], add=True)
        return carry

    lax.fori_loop(0, NCHUNK, chunk, 0)
    plsc.subcore_barrier()

    pltpu.sync_copy(agg.at[pl.ds(sid * RPS, RPS)],
                    out_hbm.at[cid, pl.ds(sid * RPS, RPS)])


_sc_aggregate = functools.partial(
    pl.kernel,
    mesh=plsc.VectorSubcoreMesh(core_axis_name="c", subcore_axis_name="s"),
    out_type=jax.ShapeDtypeStruct((NC, NP, D), jnp.float32),
    scratch_types=[
        pltpu.VMEM((CH,), jnp.int32),
        pltpu.VMEM((CH,), jnp.int32),
        pltpu.VMEM((CH, D), jnp.float32),
        pltpu.VMEM((ZROWS, D), jnp.float32),
        pltpu.VMEM_SHARED((NP, D), jnp.float32),
        pltpu.SemaphoreType.DMA,
    ],
)(_agg_body)


BR = 1000


def _tc_layer_body(p_ref, w_ref, b_ref, o_ref):
    a = p_ref[0] + p_ref[1]
    o_ref[...] = jnp.maximum(
        jnp.dot(a, w_ref[...], preferred_element_type=jnp.float32) + b_ref[...],
        0.0,
    )


def _tc_head_body(p_ref, w2_ref, b2_ref, wh_ref, bh_ref, wo_ref, bo_ref, o_ref):
    a = p_ref[0] + p_ref[1]
    h = jnp.maximum(
        jnp.dot(a, w2_ref[...], preferred_element_type=jnp.float32) + b2_ref[...],
        0.0,
    )
    h2 = jnp.maximum(
        jnp.dot(h, wh_ref[...], preferred_element_type=jnp.float32) + bh_ref[...],
        0.0,
    )
    o_ref[...] = (
        jnp.dot(h2, wo_ref[...], preferred_element_type=jnp.float32) + bo_ref[...]
    )


def _tc_layer(p, w, b):
    return pl.pallas_call(
        _tc_layer_body,
        grid=(N // BR,),
        in_specs=[
            pl.BlockSpec((NC, BR, D), lambda i: (0, i, 0)),
            pl.BlockSpec((D, D), lambda i: (0, 0)),
            pl.BlockSpec((1, D), lambda i: (0, 0)),
        ],
        out_specs=pl.BlockSpec((BR, D), lambda i: (i, 0)),
        out_shape=jax.ShapeDtypeStruct((N, D), jnp.float32),
    )(p, w, b.reshape(1, D))


def _tc_head(p, w2, b2, wh, bh, wo, bo):
    return pl.pallas_call(
        _tc_head_body,
        grid=(N // BR,),
        in_specs=[
            pl.BlockSpec((NC, BR, D), lambda i: (0, i, 0)),
            pl.BlockSpec((D, D), lambda i: (0, 0)),
            pl.BlockSpec((1, D), lambda i: (0, 0)),
            pl.BlockSpec((D, D), lambda i: (0, 0)),
            pl.BlockSpec((1, D), lambda i: (0, 0)),
            pl.BlockSpec((D, C), lambda i: (0, 0)),
            pl.BlockSpec((1, C), lambda i: (0, 0)),
        ],
        out_specs=pl.BlockSpec((BR, C), lambda i: (i, 0)),
        out_shape=jax.ShapeDtypeStruct((N, C), jnp.float32),
    )(p, w2, b2.reshape(1, D), wh, bh.reshape(1, D), wo, bo.reshape(1, C))


def kernel(x, edge_index, W1, b1, W2, b2, Wh, bh, Wo, bo):
    src = edge_index[0]
    dst = edge_index[1]
    p1 = _sc_aggregate(x, src, dst)
    h1 = _tc_layer(p1, W1, b1)
    p2 = _sc_aggregate(h1, src, dst)
    return _tc_head(p2, W2, b2, Wh, bh, Wo, bo)


# speedup vs baseline: 14.0720x; 2.7185x over previous
import functools

import jax
import jax.numpy as jnp
from jax import lax
from jax.experimental import pallas as pl
from jax.experimental.pallas import tpu as pltpu
from jax.experimental.pallas import tpu_sc as plsc

N = 10000
E = 320000
D = 128
C = 10

NC = 2
NS = 16
NW = NC * NS
EPT = E // NW
CH = 40
NCHUNK = EPT // CH
NBUF = 5
NGROUP = NCHUNK // NBUF
NP = 10240
ZROWS = 32
RPS = NP // NS


def _stage(src_hbm, dst_hbm, sidx, didx, isem, wid, g):
    p = lax.rem(g, 3)
    a = pltpu.make_async_copy(src_hbm.at[wid, g], sidx.at[p], isem.at[p])
    b = pltpu.make_async_copy(dst_hbm.at[wid, g], didx.at[p], isem.at[p])
    return a, b


def _agg_body(h_hbm, src_hbm, dst_hbm, out_hbm, sidx, didx, rows, zbuf, agg,
              gsem, isem):
    cid = lax.axis_index("c")
    sid = lax.axis_index("s")
    wid = cid * NS + sid

    pltpu.sync_copy(src_hbm.at[wid, 0], sidx.at[0])
    pltpu.sync_copy(dst_hbm.at[wid, 0], didx.at[0])
    for d in _stage(src_hbm, dst_hbm, sidx, didx, isem, wid, 1):
        d.start()

    for j in range(NBUF):
        pltpu.async_copy(h_hbm.at[sidx.at[0, j]], rows.at[j], gsem.at[j])

    def zrow(i, carry):
        for q in range(D // 16):
            zbuf[i, pl.ds(q * 16, 16)] = jnp.zeros((16,), jnp.float32)
        return carry

    lax.fori_loop(0, ZROWS, zrow, 0)
    for k in range(RPS // ZROWS):
        pltpu.sync_copy(zbuf, agg.at[pl.ds(sid * RPS + k * ZROWS, ZROWS)])
    plsc.subcore_barrier()

    def group(g, carry):
        p = lax.rem(g, 3)
        pn = lax.rem(g + 1, 3)

        @pl.when(g + 1 < NGROUP)
        def _():
            for d in _stage(src_hbm, dst_hbm, sidx, didx, isem, wid, g + 1):
                d.wait()

        @pl.when(g + 2 < NGROUP)
        def _():
            for d in _stage(src_hbm, dst_hbm, sidx, didx, isem, wid, g + 2):
                d.start()

        for j in range(NBUF):
            pltpu.make_async_copy(
                h_hbm.at[sidx.at[p, j]], rows.at[j], gsem.at[j]).wait()
            pltpu.sync_copy(rows.at[j], agg.at[didx.at[p, j]], add=True)

            @pl.when(g + 1 < NGROUP)
            def _():
                pltpu.async_copy(
                    h_hbm.at[sidx.at[pn, j]], rows.at[j], gsem.at[j])
        return carry

    lax.fori_loop(0, NGROUP, group, 0)
    plsc.subcore_barrier()

    pltpu.sync_copy(agg.at[pl.ds(sid * RPS, RPS)],
                    out_hbm.at[cid, pl.ds(sid * RPS, RPS)])


_sc_aggregate = functools.partial(
    pl.kernel,
    mesh=plsc.VectorSubcoreMesh(core_axis_name="c", subcore_axis_name="s"),
    out_type=jax.ShapeDtypeStruct((NC, NP, D), jnp.float32),
    scratch_types=[
        pltpu.VMEM((3, NBUF, CH), jnp.int32),
        pltpu.VMEM((3, NBUF, CH), jnp.int32),
        pltpu.VMEM((NBUF, CH, D), jnp.float32),
        pltpu.VMEM((ZROWS, D), jnp.float32),
        pltpu.VMEM_SHARED((NP, D), jnp.float32),
        pltpu.SemaphoreType.DMA((NBUF,)),
        pltpu.SemaphoreType.DMA((3,)),
    ],
)(_agg_body)


BR = 1000


def _tc_layer_body(p_ref, w_ref, b_ref, o_ref):
    a = p_ref[0] + p_ref[1]
    o_ref[...] = jnp.maximum(
        jnp.dot(a, w_ref[...], preferred_element_type=jnp.float32) + b_ref[...],
        0.0,
    )


def _tc_head_body(p_ref, w2_ref, b2_ref, wh_ref, bh_ref, wo_ref, bo_ref, o_ref):
    a = p_ref[0] + p_ref[1]
    h = jnp.maximum(
        jnp.dot(a, w2_ref[...], preferred_element_type=jnp.float32) + b2_ref[...],
        0.0,
    )
    h2 = jnp.maximum(
        jnp.dot(h, wh_ref[...], preferred_element_type=jnp.float32) + bh_ref[...],
        0.0,
    )
    o_ref[...] = (
        jnp.dot(h2, wo_ref[...], preferred_element_type=jnp.float32) + bo_ref[...]
    )


def _tc_layer(p, w, b):
    return pl.pallas_call(
        _tc_layer_body,
        grid=(N // BR,),
        in_specs=[
            pl.BlockSpec((NC, BR, D), lambda i: (0, i, 0)),
            pl.BlockSpec((D, D), lambda i: (0, 0)),
            pl.BlockSpec((1, D), lambda i: (0, 0)),
        ],
        out_specs=pl.BlockSpec((BR, D), lambda i: (i, 0)),
        out_shape=jax.ShapeDtypeStruct((N, D), jnp.float32),
    )(p, w, b.reshape(1, D))


def _tc_head(p, w2, b2, wh, bh, wo, bo):
    return pl.pallas_call(
        _tc_head_body,
        grid=(N // BR,),
        in_specs=[
            pl.BlockSpec((NC, BR, D), lambda i: (0, i, 0)),
            pl.BlockSpec((D, D), lambda i: (0, 0)),
            pl.BlockSpec((1, D), lambda i: (0, 0)),
            pl.BlockSpec((D, D), lambda i: (0, 0)),
            pl.BlockSpec((1, D), lambda i: (0, 0)),
            pl.BlockSpec((D, C), lambda i: (0, 0)),
            pl.BlockSpec((1, C), lambda i: (0, 0)),
        ],
        out_specs=pl.BlockSpec((BR, C), lambda i: (i, 0)),
        out_shape=jax.ShapeDtypeStruct((N, C), jnp.float32),
    )(p, w2, b2.reshape(1, D), wh, bh.reshape(1, D), wo, bo.reshape(1, C))


def kernel(x, edge_index, W1, b1, W2, b2, Wh, bh, Wo, bo):
    src = edge_index[0].reshape(NW, NGROUP, NBUF, CH)
    dst = edge_index[1].reshape(NW, NGROUP, NBUF, CH)
    p1 = _sc_aggregate(x, src, dst)
    h1 = _tc_layer(p1, W1, b1)
    p2 = _sc_aggregate(h1, src, dst)
    return _tc_head(p2, W2, b2, Wh, bh, Wo, bo)


# speedup vs baseline: 14.7800x; 1.0503x over previous
import functools

import jax
import jax.numpy as jnp
from jax import lax
from jax.experimental import pallas as pl
from jax.experimental.pallas import tpu as pltpu
from jax.experimental.pallas import tpu_sc as plsc

N = 10000
E = 320000
D = 128
C = 10

NC = 2
NS = 16
NW = NC * NS
EPT = E // NW
CH = 40
NCHUNK = EPT // CH
NBUF = 5
NGROUP = NCHUNK // NBUF
NP = 10240
ZROWS = 64
RPS = NP // NS


def _stage(ei_hbm, sidx, didx, isem, wid, g):
    p = lax.rem(g, 3)
    a = pltpu.make_async_copy(ei_hbm.at[0, wid, g], sidx.at[p], isem.at[p])
    b = pltpu.make_async_copy(ei_hbm.at[1, wid, g], didx.at[p], isem.at[p])
    return a, b


def _agg_body(h_hbm, ei_hbm, out_hbm, sidx, didx, rows, zbuf, agg,
              gsem, isem):
    cid = lax.axis_index("c")
    sid = lax.axis_index("s")
    wid = cid * NS + sid

    pltpu.sync_copy(ei_hbm.at[0, wid, 0], sidx.at[0])
    pltpu.sync_copy(ei_hbm.at[1, wid, 0], didx.at[0])
    for d in _stage(ei_hbm, sidx, didx, isem, wid, 1):
        d.start()

    for j in range(NBUF):
        pltpu.async_copy(h_hbm.at[sidx.at[0, j]], rows.at[j], gsem.at[j])

    def zrow(i, carry):
        for q in range(D // 16):
            zbuf[i, pl.ds(q * 16, 16)] = jnp.zeros((16,), jnp.float32)
        return carry

    lax.fori_loop(0, ZROWS, zrow, 0)
    for k in range(RPS // ZROWS):
        pltpu.sync_copy(zbuf, agg.at[pl.ds(sid * RPS + k * ZROWS, ZROWS)])
    plsc.subcore_barrier()

    def group(g, carry):
        p = lax.rem(g, 3)
        pn = lax.rem(g + 1, 3)

        @pl.when(g + 1 < NGROUP)
        def _():
            for d in _stage(ei_hbm, sidx, didx, isem, wid, g + 1):
                d.wait()

        @pl.when(g + 2 < NGROUP)
        def _():
            for d in _stage(ei_hbm, sidx, didx, isem, wid, g + 2):
                d.start()

        for j in range(NBUF):
            pltpu.make_async_copy(
                h_hbm.at[sidx.at[p, j]], rows.at[j], gsem.at[j]).wait()
            pltpu.sync_copy(rows.at[j], agg.at[didx.at[p, j]], add=True)

            @pl.when(g + 1 < NGROUP)
            def _():
                pltpu.async_copy(
                    h_hbm.at[sidx.at[pn, j]], rows.at[j], gsem.at[j])
        return carry

    lax.fori_loop(0, NGROUP, group, 0)
    plsc.subcore_barrier()

    pltpu.sync_copy(agg.at[pl.ds(sid * RPS, RPS)],
                    out_hbm.at[cid, pl.ds(sid * RPS, RPS)])


_sc_aggregate = functools.partial(
    pl.kernel,
    mesh=plsc.VectorSubcoreMesh(core_axis_name="c", subcore_axis_name="s"),
    out_type=jax.ShapeDtypeStruct((NC, NP, D), jnp.float32),
    scratch_types=[
        pltpu.VMEM((3, NBUF, CH), jnp.int32),
        pltpu.VMEM((3, NBUF, CH), jnp.int32),
        pltpu.VMEM((NBUF, CH, D), jnp.float32),
        pltpu.VMEM((ZROWS, D), jnp.float32),
        pltpu.VMEM_SHARED((NP, D), jnp.float32),
        pltpu.SemaphoreType.DMA((NBUF,)),
        pltpu.SemaphoreType.DMA((3,)),
    ],
)(_agg_body)


BR = 1000


def _tc_layer_body(p_ref, w_ref, b_ref, o_ref):
    a = p_ref[0] + p_ref[1]
    o_ref[...] = jnp.maximum(
        jnp.dot(a, w_ref[...], preferred_element_type=jnp.float32) + b_ref[...],
        0.0,
    )


def _tc_head_body(p_ref, w2_ref, b2_ref, wh_ref, bh_ref, wo_ref, bo_ref, o_ref):
    a = p_ref[0] + p_ref[1]
    h = jnp.maximum(
        jnp.dot(a, w2_ref[...], preferred_element_type=jnp.float32) + b2_ref[...],
        0.0,
    )
    h2 = jnp.maximum(
        jnp.dot(h, wh_ref[...], preferred_element_type=jnp.float32) + bh_ref[...],
        0.0,
    )
    o_ref[...] = (
        jnp.dot(h2, wo_ref[...], preferred_element_type=jnp.float32) + bo_ref[...]
    )


def _tc_layer(p, w, b):
    return pl.pallas_call(
        _tc_layer_body,
        grid=(N // BR,),
        in_specs=[
            pl.BlockSpec((NC, BR, D), lambda i: (0, i, 0)),
            pl.BlockSpec((D, D), lambda i: (0, 0)),
            pl.BlockSpec((1, D), lambda i: (0, 0)),
        ],
        out_specs=pl.BlockSpec((BR, D), lambda i: (i, 0)),
        out_shape=jax.ShapeDtypeStruct((N, D), jnp.float32),
    )(p, w, b.reshape(1, D))


def _tc_head(p, w2, b2, wh, bh, wo, bo):
    return pl.pallas_call(
        _tc_head_body,
        grid=(N // BR,),
        in_specs=[
            pl.BlockSpec((NC, BR, D), lambda i: (0, i, 0)),
            pl.BlockSpec((D, D), lambda i: (0, 0)),
            pl.BlockSpec((1, D), lambda i: (0, 0)),
            pl.BlockSpec((D, D), lambda i: (0, 0)),
            pl.BlockSpec((1, D), lambda i: (0, 0)),
            pl.BlockSpec((D, C), lambda i: (0, 0)),
            pl.BlockSpec((1, C), lambda i: (0, 0)),
        ],
        out_specs=pl.BlockSpec((BR, C), lambda i: (i, 0)),
        out_shape=jax.ShapeDtypeStruct((N, C), jnp.float32),
    )(p, w2, b2.reshape(1, D), wh, bh.reshape(1, D), wo, bo.reshape(1, C))


def kernel(x, edge_index, W1, b1, W2, b2, Wh, bh, Wo, bo):
    ei = edge_index.reshape(2, NW, NGROUP, NBUF, CH)
    p1 = _sc_aggregate(x, ei)
    h1 = _tc_layer(p1, W1, b1)
    p2 = _sc_aggregate(h1, ei)
    return _tc_head(p2, W2, b2, Wh, bh, Wo, bo)


# speedup vs baseline: 15.2753x; 1.0335x over previous
import functools

import jax
import jax.numpy as jnp
from jax import lax
from jax.experimental import pallas as pl
from jax.experimental.pallas import tpu as pltpu
from jax.experimental.pallas import tpu_sc as plsc

N = 10000
E = 320000
D = 128
C = 10

NC = 2
NS = 16
NW = NC * NS
EPT = E // NW
CH = 40
NCHUNK = EPT // CH
NBUF = 5
NGROUP = NCHUNK // NBUF
NP = 10240
ZROWS = 64
RPS = NP // NS


GSZ = NBUF * CH


def _stage(ei_hbm, sidx, didx, isem, wid, g):
    p = lax.rem(g, 3)
    base = wid * EPT + g * GSZ
    ds = [pltpu.make_async_copy(ei_hbm.at[pl.ds(base, GSZ)],
                                sidx.at[pl.ds(p * GSZ, GSZ)], isem.at[p])]
    for j in range(NBUF):
        ds.append(pltpu.make_async_copy(
            ei_hbm.at[pl.ds(E + base + j * CH, CH)], didx.at[p, j],
            isem.at[p]))
    return ds


def _agg_body(h_hbm, ei_hbm, out_hbm, sidx, didx, rows, zbuf, agg,
              gsem, isem):
    cid = lax.axis_index("c")
    sid = lax.axis_index("s")
    wid = cid * NS + sid

    base0 = wid * EPT
    pltpu.sync_copy(ei_hbm.at[pl.ds(base0, GSZ)], sidx.at[pl.ds(0, GSZ)])
    for j in range(NBUF):
        pltpu.sync_copy(ei_hbm.at[pl.ds(E + base0 + j * CH, CH)],
                        didx.at[0, j])
    for d in _stage(ei_hbm, sidx, didx, isem, wid, 1):
        d.start()

    for j in range(NBUF):
        pltpu.async_copy(h_hbm.at[sidx.at[pl.ds(j * CH, CH)]],
                         rows.at[j], gsem.at[j])

    def zrow(i, carry):
        for q in range(D // 16):
            zbuf[i, pl.ds(q * 16, 16)] = jnp.zeros((16,), jnp.float32)
        return carry

    lax.fori_loop(0, ZROWS, zrow, 0)
    for k in range(RPS // ZROWS):
        pltpu.sync_copy(zbuf, agg.at[pl.ds(sid * RPS + k * ZROWS, ZROWS)])
    plsc.subcore_barrier()

    def group(g, carry):
        p = lax.rem(g, 3)
        pn = lax.rem(g + 1, 3)

        @pl.when(g + 1 < NGROUP)
        def _():
            for d in _stage(ei_hbm, sidx, didx, isem, wid, g + 1):
                d.wait()

        @pl.when(g + 2 < NGROUP)
        def _():
            for d in _stage(ei_hbm, sidx, didx, isem, wid, g + 2):
                d.start()

        for j in range(NBUF):
            pltpu.make_async_copy(
                h_hbm.at[sidx.at[pl.ds(p * GSZ + j * CH, CH)]], rows.at[j],
                gsem.at[j]).wait()
            pltpu.sync_copy(rows.at[j], agg.at[didx.at[p, j]], add=True)

            @pl.when(g + 1 < NGROUP)
            def _():
                pltpu.async_copy(
                    h_hbm.at[sidx.at[pl.ds(pn * GSZ + j * CH, CH)]],
                    rows.at[j], gsem.at[j])
        return carry

    lax.fori_loop(0, NGROUP, group, 0)
    plsc.subcore_barrier()

    pltpu.sync_copy(agg.at[pl.ds(sid * RPS, RPS)],
                    out_hbm.at[cid, pl.ds(sid * RPS, RPS)])


_sc_aggregate = functools.partial(
    pl.kernel,
    mesh=plsc.VectorSubcoreMesh(core_axis_name="c", subcore_axis_name="s"),
    out_type=jax.ShapeDtypeStruct((NC, NP, D), jnp.float32),
    scratch_types=[
        pltpu.VMEM((3 * GSZ,), jnp.int32),
        pltpu.VMEM((3, NBUF, CH), jnp.int32),
        pltpu.VMEM((NBUF, CH, D), jnp.float32),
        pltpu.VMEM((ZROWS, D), jnp.float32),
        pltpu.VMEM_SHARED((NP, D), jnp.float32),
        pltpu.SemaphoreType.DMA((NBUF,)),
        pltpu.SemaphoreType.DMA((3,)),
    ],
)(_agg_body)


BR = 1000


def _tc_layer_body(p_ref, w_ref, b_ref, o_ref):
    a = p_ref[0] + p_ref[1]
    o_ref[...] = jnp.maximum(
        jnp.dot(a, w_ref[...], preferred_element_type=jnp.float32) + b_ref[...],
        0.0,
    )


def _tc_head_body(p_ref, w2_ref, b2_ref, wh_ref, bh_ref, wo_ref, bo_ref, o_ref):
    a = p_ref[0] + p_ref[1]
    h = jnp.maximum(
        jnp.dot(a, w2_ref[...], preferred_element_type=jnp.float32) + b2_ref[...],
        0.0,
    )
    h2 = jnp.maximum(
        jnp.dot(h, wh_ref[...], preferred_element_type=jnp.float32) + bh_ref[...],
        0.0,
    )
    o_ref[...] = (
        jnp.dot(h2, wo_ref[...], preferred_element_type=jnp.float32) + bo_ref[...]
    )


def _tc_layer(p, w, b):
    return pl.pallas_call(
        _tc_layer_body,
        grid=(N // BR,),
        in_specs=[
            pl.BlockSpec((NC, BR, D), lambda i: (0, i, 0)),
            pl.BlockSpec((D, D), lambda i: (0, 0)),
            pl.BlockSpec((1, D), lambda i: (0, 0)),
        ],
        out_specs=pl.BlockSpec((BR, D), lambda i: (i, 0)),
        out_shape=jax.ShapeDtypeStruct((N, D), jnp.float32),
    )(p, w, b.reshape(1, D))


def _tc_head(p, w2, b2, wh, bh, wo, bo):
    return pl.pallas_call(
        _tc_head_body,
        grid=(N // BR,),
        in_specs=[
            pl.BlockSpec((NC, BR, D), lambda i: (0, i, 0)),
            pl.BlockSpec((D, D), lambda i: (0, 0)),
            pl.BlockSpec((1, D), lambda i: (0, 0)),
            pl.BlockSpec((D, D), lambda i: (0, 0)),
            pl.BlockSpec((1, D), lambda i: (0, 0)),
            pl.BlockSpec((D, C), lambda i: (0, 0)),
            pl.BlockSpec((1, C), lambda i: (0, 0)),
        ],
        out_specs=pl.BlockSpec((BR, C), lambda i: (i, 0)),
        out_shape=jax.ShapeDtypeStruct((N, C), jnp.float32),
    )(p, w2, b2.reshape(1, D), wh, bh.reshape(1, D), wo, bo.reshape(1, C))


def kernel(x, edge_index, W1, b1, W2, b2, Wh, bh, Wo, bo):
    ei = edge_index.reshape(2 * E)
    p1 = _sc_aggregate(x, ei)
    h1 = _tc_layer(p1, W1, b1)
    p2 = _sc_aggregate(h1, ei)
    return _tc_head(p2, W2, b2, Wh, bh, Wo, bo)
